# Initial kernel scaffold; baseline (speedup 1.0000x reference)
import jax
import jax.numpy as jnp
from jax.experimental import pallas as pl


def kernel(mutation_types, features, type_table, W1, b1, W2, b2, Wf, bf, ln_gamma, ln_beta):
    raise NotImplementedError("write your pallas kernel here")


# speedup vs baseline: 1.2484x; 1.2484x over previous
import functools

import jax
import jax.numpy as jnp
from jax.experimental import pallas as pl

EMBED_DIM = 32
HALF = 16
N_FEATURES = 5
N_TYPES = 6
TBLK = 4096


def _body(mt_ref, f_ref, tt_ref, w1_ref, b1_ref, w2_ref, b2_ref, wf_ref,
          bf_ref, g_ref, beta_ref, out_ref):
    wf_top = wf_ref[0:HALF, :]
    wf_bot = wf_ref[HALF:EMBED_DIM, :]
    ft = jnp.dot(tt_ref[...], wf_top, preferred_element_type=jnp.float32)
    ft = ft + bf_ref[...]

    mt = mt_ref[...]
    type_c = jnp.zeros((mt.shape[0], EMBED_DIM), jnp.float32)
    for k in range(N_TYPES):
        mask = (mt == k).astype(jnp.float32)
        type_c = type_c + mask * ft[k:k + 1, :]

    f = f_ref[...]
    w1 = w1_ref[...]
    h = b1_ref[...]
    for j in range(N_FEATURES):
        h = h + f[:, j:j + 1] * w1[j:j + 1, :]
    h = 0.5 * h * (1.0 + jax.lax.erf(h * 0.7071067811865476))
    feat = jnp.dot(h, w2_ref[...], preferred_element_type=jnp.float32)
    feat = feat + b2_ref[...]

    out = jnp.dot(feat, wf_bot, preferred_element_type=jnp.float32) + type_c

    mu = jnp.mean(out, axis=1, keepdims=True)
    d = out - mu
    var = jnp.mean(d * d, axis=1, keepdims=True)
    y = d * jax.lax.rsqrt(var + 1e-5)
    out_ref[...] = y * g_ref[...] + beta_ref[...]


@functools.partial(jax.jit, static_argnames=())
def kernel(mutation_types, features, type_table, W1, b1, W2, b2, Wf, bf,
           ln_gamma, ln_beta):
    B, M = mutation_types.shape
    N = B * M
    mt2 = mutation_types.reshape(N, 1)
    f2 = features.reshape(N, N_FEATURES)

    small = lambda shp: pl.BlockSpec(shp, lambda i: (0,) * len(shp))
    out = pl.pallas_call(
        _body,
        grid=(N // TBLK,),
        in_specs=[
            pl.BlockSpec((TBLK, 1), lambda i: (i, 0)),
            pl.BlockSpec((TBLK, N_FEATURES), lambda i: (i, 0)),
            small((N_TYPES, HALF)),
            small((N_FEATURES, HALF)),
            small((1, HALF)),
            small((HALF, HALF)),
            small((1, HALF)),
            small((EMBED_DIM, EMBED_DIM)),
            small((1, EMBED_DIM)),
            small((1, EMBED_DIM)),
            small((1, EMBED_DIM)),
        ],
        out_specs=pl.BlockSpec((TBLK, EMBED_DIM), lambda i: (i, 0)),
        out_shape=jax.ShapeDtypeStruct((N, EMBED_DIM), jnp.float32),
    )(mt2, f2, type_table, W1, b1.reshape(1, HALF), W2, b2.reshape(1, HALF),
      Wf, bf.reshape(1, EMBED_DIM), ln_gamma.reshape(1, EMBED_DIM),
      ln_beta.reshape(1, EMBED_DIM))
    return out.reshape(B, M, EMBED_DIM)


# speedup vs baseline: 3.3585x; 2.6903x over previous
import functools

import jax
import jax.numpy as jnp
import numpy as np
from jax.experimental import pallas as pl

EMBED_DIM = 32
HALF = 16
N_FEATURES = 5
N_TYPES = 6
PACK = 4
LANES = PACK * EMBED_DIM
T4BLK = 2048


def _body(mtf_ref, fp_ref, tt_ref, p24_ref, wtopt_ref, bfp_ref, bm24_ref,
          s24_ref, mod6_ref, w1p_ref, b1p_ref, w2p_ref, b2p_ref, wfbp_ref,
          a_ref, gp_ref, bp_ref, out_ref):
    f32 = jnp.float32
    A = a_ref[...]

    tts = jnp.dot(p24_ref[...], tt_ref[...], preferred_element_type=f32)
    ftb = (jnp.dot(tts, wtopt_ref[...], preferred_element_type=f32)
           + bfp_ref[...]) * bm24_ref[...]
    ft_c = ftb - jnp.dot(ftb, A, preferred_element_type=f32)

    wfbp = wfbp_ref[...]
    wfbp_c = wfbp - jnp.dot(wfbp, A, preferred_element_type=f32)

    mt_t = jnp.dot(mtf_ref[...], s24_ref[...], preferred_element_type=f32)
    oh = (mt_t == mod6_ref[...]).astype(f32)

    h = jnp.dot(fp_ref[...], w1p_ref[...], preferred_element_type=f32)
    h = h + b1p_ref[...]
    h = 0.5 * h * (1.0 + jax.lax.erf(h * 0.7071067811865476))
    feat = jnp.dot(h, w2p_ref[...], preferred_element_type=f32)
    feat = feat + b2p_ref[...]

    d = (jnp.dot(feat, wfbp_c, preferred_element_type=f32)
         + jnp.dot(oh, ft_c, preferred_element_type=f32))

    var = jnp.dot(d * d, A, preferred_element_type=f32)
    y = d * jax.lax.rsqrt(var + 1e-5)
    out_ref[...] = y * gp_ref[...] + bp_ref[...]


def _pack_constants(W1, b1, W2, b2, Wf, bf, ln_gamma, ln_beta):
    f32 = jnp.float32
    W1p = jnp.zeros((PACK * N_FEATURES, PACK * HALF), f32)
    W2p = jnp.zeros((PACK * HALF, PACK * HALF), f32)
    Wfbp = jnp.zeros((PACK * HALF, LANES), f32)
    for g in range(PACK):
        W1p = W1p.at[g * N_FEATURES:(g + 1) * N_FEATURES,
                     g * HALF:(g + 1) * HALF].set(W1)
        W2p = W2p.at[g * HALF:(g + 1) * HALF, g * HALF:(g + 1) * HALF].set(W2)
        Wfbp = Wfbp.at[g * HALF:(g + 1) * HALF,
                       g * EMBED_DIM:(g + 1) * EMBED_DIM].set(Wf[HALF:, :])
    b1p = jnp.tile(b1, PACK).reshape(1, PACK * HALF)
    b2p = jnp.tile(b2, PACK).reshape(1, PACK * HALF)
    bfp = jnp.tile(bf, PACK).reshape(1, LANES)
    gp = jnp.tile(ln_gamma, PACK).reshape(1, LANES)
    bp = jnp.tile(ln_beta, PACK).reshape(1, LANES)
    WtopT = jnp.tile(Wf[:HALF, :], (1, PACK))

    r24 = np.arange(PACK * N_TYPES)
    P24 = jnp.asarray((r24[:, None] % N_TYPES) == np.arange(N_TYPES)[None, :],
                      f32)
    bm24 = jnp.asarray((r24[:, None] // N_TYPES) ==
                       (np.arange(LANES)[None, :] // EMBED_DIM), f32)
    S24 = jnp.asarray(np.arange(PACK)[:, None] == (r24[None, :] // N_TYPES),
                      f32)
    mod6 = jnp.asarray(r24 % N_TYPES, f32).reshape(1, PACK * N_TYPES)
    cL = np.arange(LANES)
    A = jnp.asarray((cL[:, None] // EMBED_DIM) == (cL[None, :] // EMBED_DIM),
                    f32) / EMBED_DIM
    return (P24, WtopT, bfp, bm24, S24, mod6, W1p, b1p, W2p, b2p, Wfbp, A,
            gp, bp)


@jax.jit
def kernel(mutation_types, features, type_table, W1, b1, W2, b2, Wf, bf,
           ln_gamma, ln_beta):
    B, M = mutation_types.shape
    N4 = (B * M) // PACK
    mtf = mutation_types.astype(jnp.float32).reshape(N4, PACK)
    fp = features.reshape(N4, PACK * N_FEATURES)
    consts = _pack_constants(W1, b1, W2, b2, Wf, bf, ln_gamma, ln_beta)

    small = lambda shp: pl.BlockSpec(shp, lambda i: (0,) * len(shp))
    const_specs = [small(c.shape) for c in consts]
    out = pl.pallas_call(
        _body,
        grid=(N4 // T4BLK,),
        in_specs=[
            pl.BlockSpec((T4BLK, PACK), lambda i: (i, 0)),
            pl.BlockSpec((T4BLK, PACK * N_FEATURES), lambda i: (i, 0)),
            small((N_TYPES, HALF)),
        ] + const_specs,
        out_specs=pl.BlockSpec((T4BLK, LANES), lambda i: (i, 0)),
        out_shape=jax.ShapeDtypeStruct((N4, LANES), jnp.float32),
    )(mtf, fp, type_table, *consts)
    return out.reshape(B, M, EMBED_DIM)
